# baseline (speedup 1.0000x reference)
import functools

import jax
import jax.numpy as jnp
from jax import lax
from jax.experimental import pallas as pl
from jax.experimental.pallas import tpu as pltpu
from jax.experimental.pallas import tpu_sc as plsc

N = 4096
M = 16384
D = 3
KPAD = 8
CPAD = 128
MT = 512
AW = 512
NSTEPS = M // MT


def _tc_body(z8_ref, cbt_ref, ind_ref, code_ref, accv_ref, acci_ref, x2_ref):
    i = pl.program_id(0)

    @pl.when(i == 0)
    def _init():
        accv_ref[...] = jnp.full((N, AW), jnp.inf, jnp.float32)
        acci_ref[...] = jnp.zeros((N, AW), jnp.int32)
        z0 = z8_ref[:, 0:1]
        z1 = z8_ref[:, 1:2]
        z2 = z8_ref[:, 2:3]
        x2_ref[...] = z0 * z0 + z1 * z1 + z2 * z2

    t0 = cbt_ref[0:1, :]
    t1 = cbt_ref[1:2, :]
    t2 = cbt_ref[2:3, :]
    n = jnp.maximum(jnp.sqrt(t0 * t0 + t2 * t2 + t1 * t1), 1e-12)
    co0 = t0 / n
    co1 = t1 / n
    co2 = t2 / n
    code8t = jnp.concatenate(
        [co0, co1, co2, jnp.zeros((KPAD - 3, MT), jnp.float32)], axis=0)
    code_ref[...] = jnp.concatenate(
        [jnp.transpose(code8t), jnp.zeros((MT, CPAD - KPAD), jnp.float32)],
        axis=1)
    y2 = co0 * co0 + co2 * co2 + co1 * co1

    code8t2 = jnp.concatenate(
        [co0 + co0, co1 + co1, co2 + co2,
         jnp.zeros((KPAD - 3, MT), jnp.float32)], axis=0)
    p2 = jnp.dot(z8_ref[...], code8t2, preferred_element_type=jnp.float32)

    x2 = x2_ref[...]
    accv = accv_ref[...]
    acci = acci_ref[...]
    for c in range(MT // AW):
        y2c = lax.slice(y2, (0, AW * c), (1, AW * c + AW))
        p2c = lax.slice(p2, (0, AW * c), (N, AW * c + AW))
        d = (x2 + y2c) - p2c
        sid = i * (MT // AW) + c
        pred = d < accv
        accv = jnp.minimum(accv, d)
        acci = jnp.where(pred, sid, acci)
    accv_ref[...] = accv
    acci_ref[...] = acci

    @pl.when(i == NSTEPS - 1)
    def _finish():
        m = jnp.min(accv, axis=1, keepdims=True)
        liota = lax.broadcasted_iota(jnp.int32, (N, AW), 1)
        big = jnp.int32(2**30)
        full = jnp.where(accv == m, acci * AW + liota, big)
        ind_ref[...] = jnp.min(full, axis=1, keepdims=True)


def _vq_tc(z8, cbt):
    return pl.pallas_call(
        _tc_body,
        grid=(NSTEPS,),
        in_specs=[
            pl.BlockSpec((N, KPAD), lambda i: (0, 0)),
            pl.BlockSpec((D, MT), lambda i: (0, i)),
        ],
        out_specs=[
            pl.BlockSpec((N, 1), lambda i: (0, 0)),
            pl.BlockSpec((MT, CPAD), lambda i: (i, 0)),
        ],
        out_shape=[
            jax.ShapeDtypeStruct((N, 1), jnp.int32),
            jax.ShapeDtypeStruct((M, CPAD), jnp.float32),
        ],
        scratch_shapes=[
            pltpu.VMEM((N, AW), jnp.float32),
            pltpu.VMEM((N, AW), jnp.int32),
            pltpu.VMEM((N, 1), jnp.float32),
        ],
        compiler_params=pltpu.CompilerParams(
            dimension_semantics=("arbitrary",)),
    )(z8, cbt)


def _sc_gather(code16, ind):
    info = plsc.get_sparse_core_info()
    nw = info.num_cores * info.num_subcores
    bpw = N // nw
    mesh = plsc.VectorSubcoreMesh(core_axis_name="c", subcore_axis_name="s")

    @functools.partial(
        pl.kernel,
        mesh=mesh,
        out_type=jax.ShapeDtypeStruct((N, CPAD), jnp.float32),
        scratch_types=[
            pltpu.VMEM((bpw,), jnp.int32),
            pltpu.VMEM((bpw, CPAD), jnp.float32),
            pltpu.SemaphoreType.DMA,
        ],
    )
    def gather_kernel(code_hbm, ind_hbm, out_hbm, idx_v, rows_v, sem):
        wid = lax.axis_index("s") * info.num_cores + lax.axis_index("c")
        base = wid * bpw
        pltpu.sync_copy(ind_hbm.at[pl.ds(base, bpw)], idx_v)
        pltpu.async_copy(code_hbm.at[idx_v], rows_v, sem).wait()
        pltpu.sync_copy(rows_v, out_hbm.at[pl.ds(base, bpw)])

    return gather_kernel(code16, ind)


def kernel(z, codebook):
    b, d, h, w = z.shape
    z_flat = jnp.transpose(z, (0, 2, 3, 1)).reshape(-1, d)
    z8 = jnp.pad(z_flat, ((0, 0), (0, KPAD - D)))
    ind2d, code16 = _vq_tc(z8, codebook.T)
    ind = ind2d.reshape(N)
    zq16 = _sc_gather(code16, ind)
    z_q = jnp.transpose(zq16[:, :D].reshape(b, h, w, d), (0, 3, 1, 2))
    loss = jnp.zeros([1], dtype=z_q.dtype)
    return (z_q, loss, ind)


# speedup vs baseline: 1.3367x; 1.3367x over previous
import functools

import jax
import jax.numpy as jnp
from jax import lax
from jax.experimental import pallas as pl
from jax.experimental.pallas import tpu as pltpu
from jax.experimental.pallas import tpu_sc as plsc

N = 4096
M = 16384
D = 3
KPAD = 8
CPAD = 128
MT = 512
AW = 128
NSTEPS = M // MT


def _tc_body(z8_ref, cbt_ref, ind_ref, code_ref, accv_ref, acci_ref, x2_ref):
    i = pl.program_id(0)

    @pl.when(i == 0)
    def _init():
        accv_ref[...] = jnp.full((N, AW), jnp.inf, jnp.float32)
        acci_ref[...] = jnp.zeros((N, AW), jnp.int32)
        z0 = z8_ref[:, 0:1]
        z1 = z8_ref[:, 1:2]
        z2 = z8_ref[:, 2:3]
        x2_ref[...] = z0 * z0 + z1 * z1 + z2 * z2

    t0 = cbt_ref[0:1, :]
    t1 = cbt_ref[1:2, :]
    t2 = cbt_ref[2:3, :]
    n = jnp.maximum(jnp.sqrt(t0 * t0 + t2 * t2 + t1 * t1), 1e-12)
    co0 = t0 / n
    co1 = t1 / n
    co2 = t2 / n
    code8t = jnp.concatenate(
        [co0, co1, co2, jnp.zeros((KPAD - 3, MT), jnp.float32)], axis=0)
    code_ref[:, 0:KPAD] = jnp.transpose(code8t)
    y2 = co0 * co0 + co2 * co2 + co1 * co1

    code8t2 = jnp.concatenate(
        [co0 + co0, co1 + co1, co2 + co2,
         jnp.zeros((KPAD - 3, MT), jnp.float32)], axis=0)
    p2 = jnp.dot(z8_ref[...], code8t2, preferred_element_type=jnp.float32)

    x2 = x2_ref[...]
    accv = accv_ref[...]
    acci = acci_ref[...]
    for c in range(MT // AW):
        y2c = lax.slice(y2, (0, AW * c), (1, AW * c + AW))
        p2c = lax.slice(p2, (0, AW * c), (N, AW * c + AW))
        d = (x2 + y2c) - p2c
        sid = i * (MT // AW) + c
        pred = d < accv
        accv = jnp.minimum(accv, d)
        acci = jnp.where(pred, sid, acci)
    accv_ref[...] = accv
    acci_ref[...] = acci

    @pl.when(i == NSTEPS - 1)
    def _finish():
        m = jnp.min(accv, axis=1, keepdims=True)
        liota = lax.broadcasted_iota(jnp.int32, (N, AW), 1)
        big = jnp.int32(2**30)
        full = jnp.where(accv == m, acci * AW + liota, big)
        ind_ref[...] = jnp.min(full, axis=1, keepdims=True)


def _vq_tc(z8, cbt):
    return pl.pallas_call(
        _tc_body,
        grid=(NSTEPS,),
        in_specs=[
            pl.BlockSpec((N, KPAD), lambda i: (0, 0)),
            pl.BlockSpec((D, MT), lambda i: (0, i)),
        ],
        out_specs=[
            pl.BlockSpec((N, 1), lambda i: (0, 0)),
            pl.BlockSpec((MT, CPAD), lambda i: (i, 0)),
        ],
        out_shape=[
            jax.ShapeDtypeStruct((N, 1), jnp.int32),
            jax.ShapeDtypeStruct((M, CPAD), jnp.float32),
        ],
        scratch_shapes=[
            pltpu.VMEM((N, AW), jnp.float32),
            pltpu.VMEM((N, AW), jnp.int32),
            pltpu.VMEM((N, 1), jnp.float32),
        ],
        compiler_params=pltpu.CompilerParams(
            dimension_semantics=("arbitrary",)),
    )(z8, cbt)


def _sc_gather(code16, ind):
    info = plsc.get_sparse_core_info()
    nw = info.num_cores * info.num_subcores
    bpw = N // nw
    mesh = plsc.VectorSubcoreMesh(core_axis_name="c", subcore_axis_name="s")

    @functools.partial(
        pl.kernel,
        mesh=mesh,
        out_type=jax.ShapeDtypeStruct((N, CPAD), jnp.float32),
        scratch_types=[
            pltpu.VMEM((bpw,), jnp.int32),
            pltpu.VMEM((bpw, CPAD), jnp.float32),
            pltpu.SemaphoreType.DMA,
        ],
    )
    def gather_kernel(code_hbm, ind_hbm, out_hbm, idx_v, rows_v, sem):
        wid = lax.axis_index("s") * info.num_cores + lax.axis_index("c")
        base = wid * bpw
        pltpu.sync_copy(ind_hbm.at[pl.ds(base, bpw)], idx_v)
        pltpu.async_copy(code_hbm.at[idx_v], rows_v, sem).wait()
        pltpu.sync_copy(rows_v, out_hbm.at[pl.ds(base, bpw)])

    return gather_kernel(code16, ind)


def kernel(z, codebook):
    b, d, h, w = z.shape
    z_flat = jnp.transpose(z, (0, 2, 3, 1)).reshape(-1, d)
    z8 = jnp.pad(z_flat, ((0, 0), (0, KPAD - D)))
    ind2d, code16 = _vq_tc(z8, codebook.T)
    ind = ind2d.reshape(N)
    zq16 = _sc_gather(code16, ind)
    z_q = jnp.transpose(zq16[:, :D].reshape(b, h, w, d), (0, 3, 1, 2))
    loss = jnp.zeros([1], dtype=z_q.dtype)
    return (z_q, loss, ind)


# speedup vs baseline: 1.3917x; 1.0412x over previous
import functools

import jax
import jax.numpy as jnp
from jax import lax
from jax.experimental import pallas as pl
from jax.experimental.pallas import tpu as pltpu
from jax.experimental.pallas import tpu_sc as plsc

N = 4096
M = 16384
D = 3
KPAD = 8
CPAD = 128
MT = 1024
AW = 128
NSTEPS = M // MT


def _tc_body(z8_ref, cbt_ref, ind_ref, code_ref, accv_ref, acci_ref, x2_ref):
    i = pl.program_id(0)

    @pl.when(i == 0)
    def _init():
        accv_ref[...] = jnp.full((N, AW), jnp.inf, jnp.float32)
        acci_ref[...] = jnp.zeros((N, AW), jnp.int32)
        z0 = z8_ref[:, 0:1]
        z1 = z8_ref[:, 1:2]
        z2 = z8_ref[:, 2:3]
        x2_ref[...] = z0 * z0 + z1 * z1 + z2 * z2

    t0 = cbt_ref[0:1, :]
    t1 = cbt_ref[1:2, :]
    t2 = cbt_ref[2:3, :]
    n = jnp.maximum(jnp.sqrt(t0 * t0 + t2 * t2 + t1 * t1), 1e-12)
    co0 = t0 / n
    co1 = t1 / n
    co2 = t2 / n
    code8t = jnp.concatenate(
        [co0, co1, co2, jnp.zeros((KPAD - 3, MT), jnp.float32)], axis=0)
    code_ref[:, 0:KPAD] = jnp.transpose(code8t)
    y2 = co0 * co0 + co2 * co2 + co1 * co1

    code8t2 = jnp.concatenate(
        [co0 + co0, co1 + co1, co2 + co2,
         jnp.zeros((KPAD - 3, MT), jnp.float32)], axis=0)
    p2 = jnp.dot(z8_ref[...], code8t2, preferred_element_type=jnp.float32)

    x2 = x2_ref[...]
    accv = accv_ref[...]
    acci = acci_ref[...]
    for c in range(MT // AW):
        y2c = lax.slice(y2, (0, AW * c), (1, AW * c + AW))
        p2c = lax.slice(p2, (0, AW * c), (N, AW * c + AW))
        d = (x2 + y2c) - p2c
        sid = i * (MT // AW) + c
        pred = d < accv
        accv = jnp.minimum(accv, d)
        acci = jnp.where(pred, sid, acci)
    accv_ref[...] = accv
    acci_ref[...] = acci

    @pl.when(i == NSTEPS - 1)
    def _finish():
        m = jnp.min(accv, axis=1, keepdims=True)
        liota = lax.broadcasted_iota(jnp.int32, (N, AW), 1)
        big = jnp.int32(2**30)
        full = jnp.where(accv == m, acci * AW + liota, big)
        ind_ref[...] = jnp.min(full, axis=1, keepdims=True)


def _vq_tc(z8, cbt):
    return pl.pallas_call(
        _tc_body,
        grid=(NSTEPS,),
        in_specs=[
            pl.BlockSpec((N, KPAD), lambda i: (0, 0)),
            pl.BlockSpec((D, MT), lambda i: (0, i)),
        ],
        out_specs=[
            pl.BlockSpec((N, 1), lambda i: (0, 0)),
            pl.BlockSpec((MT, CPAD), lambda i: (i, 0)),
        ],
        out_shape=[
            jax.ShapeDtypeStruct((N, 1), jnp.int32),
            jax.ShapeDtypeStruct((M, CPAD), jnp.float32),
        ],
        scratch_shapes=[
            pltpu.VMEM((N, AW), jnp.float32),
            pltpu.VMEM((N, AW), jnp.int32),
            pltpu.VMEM((N, 1), jnp.float32),
        ],
        compiler_params=pltpu.CompilerParams(
            dimension_semantics=("arbitrary",)),
    )(z8, cbt)


def _sc_gather(code16, ind):
    info = plsc.get_sparse_core_info()
    nw = info.num_cores * info.num_subcores
    bpw = N // nw
    mesh = plsc.VectorSubcoreMesh(core_axis_name="c", subcore_axis_name="s")

    @functools.partial(
        pl.kernel,
        mesh=mesh,
        out_type=jax.ShapeDtypeStruct((N, CPAD), jnp.float32),
        scratch_types=[
            pltpu.VMEM((bpw,), jnp.int32),
            pltpu.VMEM((bpw, CPAD), jnp.float32),
            pltpu.SemaphoreType.DMA,
        ],
    )
    def gather_kernel(code_hbm, ind_hbm, out_hbm, idx_v, rows_v, sem):
        wid = lax.axis_index("s") * info.num_cores + lax.axis_index("c")
        base = wid * bpw
        pltpu.sync_copy(ind_hbm.at[pl.ds(base, bpw)], idx_v)
        pltpu.async_copy(code_hbm.at[idx_v], rows_v, sem).wait()
        pltpu.sync_copy(rows_v, out_hbm.at[pl.ds(base, bpw)])

    return gather_kernel(code16, ind)


def kernel(z, codebook):
    b, d, h, w = z.shape
    z_flat = jnp.transpose(z, (0, 2, 3, 1)).reshape(-1, d)
    z8 = jnp.pad(z_flat, ((0, 0), (0, KPAD - D)))
    ind2d, code16 = _vq_tc(z8, codebook.T)
    ind = ind2d.reshape(N)
    zq16 = _sc_gather(code16, ind)
    z_q = jnp.transpose(zq16[:, :D].reshape(b, h, w, d), (0, 3, 1, 2))
    loss = jnp.zeros([1], dtype=z_q.dtype)
    return (z_q, loss, ind)


# speedup vs baseline: 1.4330x; 1.0296x over previous
import functools

import jax
import jax.numpy as jnp
from jax import lax
from jax.experimental import pallas as pl
from jax.experimental.pallas import tpu as pltpu
from jax.experimental.pallas import tpu_sc as plsc

N = 4096
M = 16384
D = 3
KPAD = 8
CPAD = 128
MT = 2048
AW = 128
NSTEPS = M // MT


def _tc_body(z8_ref, cbt_ref, ind_ref, code_ref, accv_ref, acci_ref, x2_ref):
    i = pl.program_id(0)

    @pl.when(i == 0)
    def _init():
        accv_ref[...] = jnp.full((N, AW), jnp.inf, jnp.float32)
        acci_ref[...] = jnp.zeros((N, AW), jnp.int32)
        z0 = z8_ref[:, 0:1]
        z1 = z8_ref[:, 1:2]
        z2 = z8_ref[:, 2:3]
        x2_ref[...] = z0 * z0 + z1 * z1 + z2 * z2

    t0 = cbt_ref[0:1, :]
    t1 = cbt_ref[1:2, :]
    t2 = cbt_ref[2:3, :]
    n = jnp.maximum(jnp.sqrt(t0 * t0 + t2 * t2 + t1 * t1), 1e-12)
    co0 = t0 / n
    co1 = t1 / n
    co2 = t2 / n
    code8t = jnp.concatenate(
        [co0, co1, co2, jnp.zeros((KPAD - 3, MT), jnp.float32)], axis=0)
    code_ref[:, 0:KPAD] = jnp.transpose(code8t)
    y2 = co0 * co0 + co2 * co2 + co1 * co1

    code8t2 = jnp.concatenate(
        [co0 + co0, co1 + co1, co2 + co2,
         jnp.zeros((KPAD - 3, MT), jnp.float32)], axis=0)
    p2 = jnp.dot(z8_ref[...], code8t2, preferred_element_type=jnp.float32)

    x2 = x2_ref[...]
    accv = accv_ref[...]
    acci = acci_ref[...]
    for c in range(MT // AW):
        y2c = lax.slice(y2, (0, AW * c), (1, AW * c + AW))
        p2c = lax.slice(p2, (0, AW * c), (N, AW * c + AW))
        d = (x2 + y2c) - p2c
        sid = i * (MT // AW) + c
        pred = d < accv
        accv = jnp.minimum(accv, d)
        acci = jnp.where(pred, sid, acci)
    accv_ref[...] = accv
    acci_ref[...] = acci

    @pl.when(i == NSTEPS - 1)
    def _finish():
        m = jnp.min(accv, axis=1, keepdims=True)
        liota = lax.broadcasted_iota(jnp.int32, (N, AW), 1)
        big = jnp.int32(2**30)
        full = jnp.where(accv == m, acci * AW + liota, big)
        ind_ref[...] = jnp.min(full, axis=1, keepdims=True)


def _vq_tc(z8, cbt):
    return pl.pallas_call(
        _tc_body,
        grid=(NSTEPS,),
        in_specs=[
            pl.BlockSpec((N, KPAD), lambda i: (0, 0)),
            pl.BlockSpec((D, MT), lambda i: (0, i)),
        ],
        out_specs=[
            pl.BlockSpec((N, 1), lambda i: (0, 0)),
            pl.BlockSpec((MT, CPAD), lambda i: (i, 0)),
        ],
        out_shape=[
            jax.ShapeDtypeStruct((N, 1), jnp.int32),
            jax.ShapeDtypeStruct((M, CPAD), jnp.float32),
        ],
        scratch_shapes=[
            pltpu.VMEM((N, AW), jnp.float32),
            pltpu.VMEM((N, AW), jnp.int32),
            pltpu.VMEM((N, 1), jnp.float32),
        ],
        compiler_params=pltpu.CompilerParams(
            dimension_semantics=("arbitrary",)),
    )(z8, cbt)


def _sc_gather(code16, ind):
    info = plsc.get_sparse_core_info()
    nw = info.num_cores * info.num_subcores
    bpw = N // nw
    mesh = plsc.VectorSubcoreMesh(core_axis_name="c", subcore_axis_name="s")

    @functools.partial(
        pl.kernel,
        mesh=mesh,
        out_type=jax.ShapeDtypeStruct((N, CPAD), jnp.float32),
        scratch_types=[
            pltpu.VMEM((bpw,), jnp.int32),
            pltpu.VMEM((bpw, CPAD), jnp.float32),
            pltpu.SemaphoreType.DMA,
        ],
    )
    def gather_kernel(code_hbm, ind_hbm, out_hbm, idx_v, rows_v, sem):
        wid = lax.axis_index("s") * info.num_cores + lax.axis_index("c")
        base = wid * bpw
        pltpu.sync_copy(ind_hbm.at[pl.ds(base, bpw)], idx_v)
        pltpu.async_copy(code_hbm.at[idx_v], rows_v, sem).wait()
        pltpu.sync_copy(rows_v, out_hbm.at[pl.ds(base, bpw)])

    return gather_kernel(code16, ind)


def kernel(z, codebook):
    b, d, h, w = z.shape
    z_flat = jnp.transpose(z, (0, 2, 3, 1)).reshape(-1, d)
    z8 = jnp.pad(z_flat, ((0, 0), (0, KPAD - D)))
    ind2d, code16 = _vq_tc(z8, codebook.T)
    ind = ind2d.reshape(N)
    zq16 = _sc_gather(code16, ind)
    z_q = jnp.transpose(zq16[:, :D].reshape(b, h, w, d), (0, 3, 1, 2))
    loss = jnp.zeros([1], dtype=z_q.dtype)
    return (z_q, loss, ind)
